# baseline (speedup 1.0000x reference)
import functools

import jax
import jax.numpy as jnp
from jax import lax
from jax.experimental import pallas as pl
from jax.experimental.pallas import tpu as pltpu

N = 10000
M = 5000
BLK = 400
BLK_BC = 1000

_F32 = jnp.float32
_BF16 = jnp.bfloat16


def _tdot(a, b):
    return lax.dot_general(
        a, b, (((0,), (0,)), ((), ())), preferred_element_type=_F32
    )


def _pass_a(h_ref, x_ref, w1_ref, b1_ref, a1_ref, de_ref, dvis_ref, hb_ref):
    i = pl.program_id(0)
    hb = h_ref[...]
    dv = jnp.sum(hb, axis=1)
    dvis = 1.0 / jnp.sqrt(dv + 1e-12)
    dvis_ref[...] = dvis[:, None]
    hbb = hb.astype(_BF16)
    hb_ref[...] = (
        jnp.clip(jnp.floor(hb * 256.0), 0.0, 255.0) - 128.0
    ).astype(jnp.int8)
    x1 = (
        jnp.dot(x_ref[...], w1_ref[...], preferred_element_type=_F32)
        + b1_ref[...]
    )
    xs = (dvis[:, None] * x1).astype(_BF16)
    contrib = _tdot(hbb, xs)
    de_part = jnp.sum(hb, axis=0)[:, None]

    @pl.when(i == 0)
    def _():
        a1_ref[...] = contrib
        de_ref[...] = de_part

    @pl.when(i != 0)
    def _():
        a1_ref[...] += contrib
        de_ref[...] += de_part


def _pass_b(h_ref, dvis_ref, a1_ref, de_ref, w2_ref, b2_ref, a2_ref):
    i = pl.program_id(0)
    de_inv = 1.0 / (de_ref[...] + 1e-12)
    a1s = (de_inv * a1_ref[...]).astype(_BF16)
    hb = ((h_ref[...].astype(_F32) + 128.5) * (1.0 / 256.0)).astype(_BF16)
    b1 = jnp.dot(hb, a1s, preferred_element_type=_F32)
    dvis = dvis_ref[...]
    x1o = jnp.maximum(dvis * b1, 0.0)
    z = dvis * (
        jnp.dot(x1o, w2_ref[...], preferred_element_type=_F32) + b2_ref[...]
    )
    contrib = _tdot(hb, z.astype(_BF16))

    @pl.when(i == 0)
    def _():
        a2_ref[...] = contrib

    @pl.when(i != 0)
    def _():
        a2_ref[...] += contrib


def _pass_c(h_ref, dvis_ref, a2_ref, de_ref, out_ref):
    hb = ((h_ref[...].astype(_F32) + 128.5) * (1.0 / 256.0)).astype(_BF16)
    de_inv = 1.0 / (de_ref[...] + 1e-12)
    a2s = (de_inv * a2_ref[...]).astype(_BF16)
    b2 = jnp.dot(hb, a2s, preferred_element_type=_F32)
    out_ref[...] = dvis_ref[...] * b2


@functools.partial(jax.jit, static_argnames=("interpret",))
def _run(X, H, W1, b1, W2, b2, interpret=False):
    n, c_in = X.shape
    m = H.shape[1]
    c_hid = W1.shape[1]
    c_out = W2.shape[1]
    nblk = n // BLK
    b1r = b1.reshape(1, c_hid)
    b2r = b2.reshape(1, c_out)

    nblk_bc = n // BLK_BC
    grid = (nblk,)
    grid_bc = (nblk_bc,)
    arb = pltpu.CompilerParams(
        dimension_semantics=("arbitrary",),
    )

    a1, de, dvis, hbf = pl.pallas_call(
        _pass_a,
        grid=grid,
        in_specs=[
            pl.BlockSpec((BLK, m), lambda i: (i, 0)),
            pl.BlockSpec((BLK, c_in), lambda i: (i, 0)),
            pl.BlockSpec((c_in, c_hid), lambda i: (0, 0)),
            pl.BlockSpec((1, c_hid), lambda i: (0, 0)),
        ],
        out_specs=[
            pl.BlockSpec((m, c_hid), lambda i: (0, 0)),
            pl.BlockSpec((m, 1), lambda i: (0, 0)),
            pl.BlockSpec((BLK, 1), lambda i: (i, 0)),
            pl.BlockSpec((BLK, m), lambda i: (i, 0)),
        ],
        out_shape=[
            jax.ShapeDtypeStruct((m, c_hid), _F32),
            jax.ShapeDtypeStruct((m, 1), _F32),
            jax.ShapeDtypeStruct((n, 1), _F32),
            jax.ShapeDtypeStruct((n, m), jnp.int8),
        ],
        compiler_params=arb,
        interpret=interpret,
    )(H, X, W1, b1r)

    a2 = pl.pallas_call(
        _pass_b,
        grid=grid_bc,
        in_specs=[
            pl.BlockSpec((BLK_BC, m), lambda i: (i, 0)),
            pl.BlockSpec((BLK_BC, 1), lambda i: (i, 0)),
            pl.BlockSpec((m, c_hid), lambda i: (0, 0)),
            pl.BlockSpec((m, 1), lambda i: (0, 0)),
            pl.BlockSpec((c_hid, c_out), lambda i: (0, 0)),
            pl.BlockSpec((1, c_out), lambda i: (0, 0)),
        ],
        out_specs=pl.BlockSpec((m, c_out), lambda i: (0, 0)),
        out_shape=jax.ShapeDtypeStruct((m, c_out), _F32),
        compiler_params=arb,
        interpret=interpret,
    )(hbf, dvis, a1, de, W2, b2r)

    out = pl.pallas_call(
        _pass_c,
        grid=grid_bc,
        in_specs=[
            pl.BlockSpec((BLK_BC, m), lambda i: (i, 0)),
            pl.BlockSpec((BLK_BC, 1), lambda i: (i, 0)),
            pl.BlockSpec((m, c_out), lambda i: (0, 0)),
            pl.BlockSpec((m, 1), lambda i: (0, 0)),
        ],
        out_specs=pl.BlockSpec((BLK_BC, c_out), lambda i: (i, 0)),
        out_shape=jax.ShapeDtypeStruct((n, c_out), _F32),
        compiler_params=arb,
        interpret=interpret,
    )(hbf, dvis, a2, de)

    return out


def kernel(X, H, W1, b1, W2, b2):
    return _run(X, H, W1, b1, W2, b2)


# speedup vs baseline: 1.0443x; 1.0443x over previous
import functools

import jax
import jax.numpy as jnp
from jax import lax
from jax.experimental import pallas as pl
from jax.experimental.pallas import tpu as pltpu

N = 10000
M = 5000
BLK = 400
BLK_BC = 1000

_F32 = jnp.float32
_BF16 = jnp.bfloat16


def _tdot(a, b):
    return lax.dot_general(
        a, b, (((0,), (0,)), ((), ())), preferred_element_type=_F32
    )


def _pass_a(h_ref, x_ref, w1_ref, b1_ref, a1_ref, de_ref, dvis_ref, hb_ref):
    i = pl.program_id(0)
    hb = h_ref[...]
    dv = jnp.sum(hb, axis=1)
    dvis = 1.0 / jnp.sqrt(dv + 1e-12)
    dvis_ref[...] = dvis[:, None]
    hbb = hb.astype(_BF16)
    hb_ref[...] = hbb
    x1 = (
        jnp.dot(x_ref[...], w1_ref[...], preferred_element_type=_F32)
        + b1_ref[...]
    )
    xs = (dvis[:, None] * x1).astype(_BF16)
    contrib = _tdot(hbb, xs)
    de_part = jnp.sum(hb, axis=0)[:, None]

    @pl.when(i == 0)
    def _():
        a1_ref[...] = contrib
        de_ref[...] = de_part

    @pl.when(i != 0)
    def _():
        a1_ref[...] += contrib
        de_ref[...] += de_part


def _pass_b(h_ref, dvis_ref, a1_ref, de_ref, w2_ref, b2_ref, a2_ref):
    i = pl.program_id(0)
    de_inv = 1.0 / (de_ref[...] + 1e-12)
    a1s = (de_inv * a1_ref[...]).astype(_BF16)
    b1 = jnp.dot(h_ref[...], a1s, preferred_element_type=_F32)
    dvis = dvis_ref[...]
    x1o = jnp.maximum(dvis * b1, 0.0)
    z = dvis * (
        jnp.dot(x1o, w2_ref[...], preferred_element_type=_F32) + b2_ref[...]
    )
    contrib = _tdot(h_ref[...], z.astype(_BF16))

    @pl.when(i == 0)
    def _():
        a2_ref[...] = contrib

    @pl.when(i != 0)
    def _():
        a2_ref[...] += contrib


def _pass_c(h_ref, dvis_ref, a2_ref, de_ref, out_ref):
    hb = h_ref[...]
    de_inv = 1.0 / (de_ref[...] + 1e-12)
    a2s = (de_inv * a2_ref[...]).astype(_BF16)
    b2 = jnp.dot(hb, a2s, preferred_element_type=_F32)
    out_ref[...] = dvis_ref[...] * b2


@functools.partial(jax.jit, static_argnames=("interpret",))
def _run(X, H, W1, b1, W2, b2, interpret=False):
    n, c_in = X.shape
    m = H.shape[1]
    c_hid = W1.shape[1]
    c_out = W2.shape[1]
    nblk = n // BLK
    b1r = b1.reshape(1, c_hid)
    b2r = b2.reshape(1, c_out)

    nblk_bc = n // BLK_BC
    grid = (nblk,)
    grid_bc = (nblk_bc,)
    arb = pltpu.CompilerParams(
        dimension_semantics=("arbitrary",),
    )

    a1, de, dvis, hbf = pl.pallas_call(
        _pass_a,
        grid=grid,
        in_specs=[
            pl.BlockSpec((BLK, m), lambda i: (i, 0)),
            pl.BlockSpec((BLK, c_in), lambda i: (i, 0)),
            pl.BlockSpec((c_in, c_hid), lambda i: (0, 0)),
            pl.BlockSpec((1, c_hid), lambda i: (0, 0)),
        ],
        out_specs=[
            pl.BlockSpec((m, c_hid), lambda i: (0, 0)),
            pl.BlockSpec((m, 1), lambda i: (0, 0)),
            pl.BlockSpec((BLK, 1), lambda i: (i, 0)),
            pl.BlockSpec((BLK, m), lambda i: (i, 0)),
        ],
        out_shape=[
            jax.ShapeDtypeStruct((m, c_hid), _F32),
            jax.ShapeDtypeStruct((m, 1), _F32),
            jax.ShapeDtypeStruct((n, 1), _F32),
            jax.ShapeDtypeStruct((n, m), _BF16),
        ],
        compiler_params=arb,
        interpret=interpret,
    )(H, X, W1, b1r)

    a2 = pl.pallas_call(
        _pass_b,
        grid=grid_bc,
        in_specs=[
            pl.BlockSpec((BLK_BC, m), lambda i: (i, 0)),
            pl.BlockSpec((BLK_BC, 1), lambda i: (i, 0)),
            pl.BlockSpec((m, c_hid), lambda i: (0, 0)),
            pl.BlockSpec((m, 1), lambda i: (0, 0)),
            pl.BlockSpec((c_hid, c_out), lambda i: (0, 0)),
            pl.BlockSpec((1, c_out), lambda i: (0, 0)),
        ],
        out_specs=pl.BlockSpec((m, c_out), lambda i: (0, 0)),
        out_shape=jax.ShapeDtypeStruct((m, c_out), _F32),
        compiler_params=arb,
        interpret=interpret,
    )(hbf, dvis, a1, de, W2, b2r)

    out = pl.pallas_call(
        _pass_c,
        grid=grid_bc,
        in_specs=[
            pl.BlockSpec((BLK_BC, m), lambda i: (i, 0)),
            pl.BlockSpec((BLK_BC, 1), lambda i: (i, 0)),
            pl.BlockSpec((m, c_out), lambda i: (0, 0)),
            pl.BlockSpec((m, 1), lambda i: (0, 0)),
        ],
        out_specs=pl.BlockSpec((BLK_BC, c_out), lambda i: (i, 0)),
        out_shape=jax.ShapeDtypeStruct((n, c_out), _F32),
        compiler_params=arb,
        interpret=interpret,
    )(hbf, dvis, a2, de)

    return out


def kernel(X, H, W1, b1, W2, b2):
    return _run(X, H, W1, b1, W2, b2)
